# baseline (speedup 1.0000x reference)
import functools

import jax
import jax.numpy as jnp
from jax import lax
from jax.experimental import pallas as pl
from jax.experimental.pallas import tpu as pltpu
from jax.experimental.pallas import tpu_sc as plsc

_NUM_CORES = 2
_NUM_SUBCORES = 16
_NUM_WORKERS = _NUM_CORES * _NUM_SUBCORES
_LANES = 16
_BATCH = 16384
_EMB_DIM = 64
_B_PER_W = _BATCH // _NUM_WORKERS
_IDX_CHUNK = 128
_N_CHUNKS = _B_PER_W // _IDX_CHUNK
_GROUPS = _B_PER_W // _LANES


def _body(wids_hbm, cids_hbm, emb_hbm, bias_hbm, fc_hbm, out_hbm,
          widx, cidx, wrows, crows, wbias, cbias, fcv, outv, sem):
    cid = lax.axis_index("c")
    sid = lax.axis_index("s")
    wid = sid * _NUM_CORES + cid
    base = wid * _B_PER_W

    pltpu.sync_copy(wids_hbm.at[pl.ds(base, _B_PER_W)], widx)
    pltpu.sync_copy(cids_hbm.at[pl.ds(base, _B_PER_W)], cidx)
    pltpu.sync_copy(fc_hbm, fcv)

    copies = []
    for j in range(_N_CHUNKS):
        sl = pl.ds(j * _IDX_CHUNK, _IDX_CHUNK)
        copies.append(pltpu.async_copy(emb_hbm.at[widx.at[sl]], wrows.at[sl], sem))
        copies.append(pltpu.async_copy(emb_hbm.at[cidx.at[sl]], crows.at[sl], sem))
        copies.append(pltpu.async_copy(bias_hbm.at[widx.at[sl]], wbias.at[sl], sem))
        copies.append(pltpu.async_copy(bias_hbm.at[cidx.at[sl]], cbias.at[sl], sem))
    for cp in copies:
        cp.wait()

    fc0 = fcv[pl.ds(0, 16)]
    fc1 = fcv[pl.ds(16, 16)]
    fc2 = fcv[pl.ds(32, 16)]
    fc3 = fcv[pl.ds(48, 16)]
    lane = lax.iota(jnp.int32, 16)

    def group(g, carry):
        acc = jnp.zeros((_LANES,), jnp.float32)
        for r in range(_LANES):
            i = g * _LANES + r
            p = wrows[i, pl.ds(0, 16)] * crows[i, pl.ds(0, 16)] * fc0
            p = p + wrows[i, pl.ds(16, 16)] * crows[i, pl.ds(16, 16)] * fc1
            p = p + wrows[i, pl.ds(32, 16)] * crows[i, pl.ds(32, 16)] * fc2
            p = p + wrows[i, pl.ds(48, 16)] * crows[i, pl.ds(48, 16)] * fc3
            s = jnp.sum(p)
            acc = jnp.where(lane == r, s, acc)
        sl = pl.ds(g * _LANES, _LANES)
        z = acc + wbias[sl] + cbias[sl]
        outv[sl] = 1.0 / (1.0 + jnp.exp(-z))
        return carry

    lax.fori_loop(0, _GROUPS, group, 0)
    pltpu.sync_copy(outv, out_hbm.at[pl.ds(base, _B_PER_W)])


@jax.jit
def _run(word_ids, context_ids, emb_table, bias_flat, fc_flat):
    mesh = plsc.VectorSubcoreMesh(core_axis_name="c", subcore_axis_name="s")
    call = functools.partial(
        pl.kernel,
        mesh=mesh,
        compiler_params=pltpu.CompilerParams(
            needs_layout_passes=False, use_tc_tiling_on_sc=False),
        out_type=jax.ShapeDtypeStruct((_BATCH,), jnp.float32),
        scratch_types=[
            pltpu.VMEM((_B_PER_W,), jnp.int32),
            pltpu.VMEM((_B_PER_W,), jnp.int32),
            pltpu.VMEM((_B_PER_W, _EMB_DIM), jnp.float32),
            pltpu.VMEM((_B_PER_W, _EMB_DIM), jnp.float32),
            pltpu.VMEM((_B_PER_W,), jnp.float32),
            pltpu.VMEM((_B_PER_W,), jnp.float32),
            pltpu.VMEM((_EMB_DIM,), jnp.float32),
            pltpu.VMEM((_B_PER_W,), jnp.float32),
            pltpu.SemaphoreType.DMA,
        ],
    )(_body)
    return call(word_ids, context_ids, emb_table, bias_flat, fc_flat)


def kernel(word_ids, context_ids, emb_table, bias_table, fc_weight):
    word_ids = word_ids.astype(jnp.int32)
    context_ids = context_ids.astype(jnp.int32)
    bias_flat = bias_table.reshape(-1)
    fc_flat = fc_weight.reshape(-1)
    out = _run(word_ids, context_ids, emb_table, bias_flat, fc_flat)
    return out.reshape(_BATCH, 1)


# speedup vs baseline: 1.5499x; 1.5499x over previous
import functools

import jax
import jax.numpy as jnp
from jax import lax
from jax.experimental import pallas as pl
from jax.experimental.pallas import tpu as pltpu
from jax.experimental.pallas import tpu_sc as plsc

_NUM_CORES = 2
_NUM_SUBCORES = 16
_NUM_WORKERS = _NUM_CORES * _NUM_SUBCORES
_LANES = 16
_BATCH = 16384
_EMB_DIM = 64
_B_PER_W = _BATCH // _NUM_WORKERS
_IDX_CHUNK = 128
_N_CHUNKS = _B_PER_W // _IDX_CHUNK
_GROUPS = _B_PER_W // _LANES
_RING = 16


def _dot_body(wids_hbm, cids_hbm, emb_hbm, fc_hbm, out_hbm,
              widx, cidx, wrows, crows, fcv, outv, sem):
    cid = lax.axis_index("c")
    sid = lax.axis_index("s")
    wid = sid * _NUM_CORES + cid
    base = wid * _B_PER_W

    pltpu.sync_copy(wids_hbm.at[pl.ds(base, _B_PER_W)], widx)
    pltpu.sync_copy(cids_hbm.at[pl.ds(base, _B_PER_W)], cidx)
    pltpu.sync_copy(fc_hbm, fcv)

    fc0 = fcv[pl.ds(0, 16)]
    fc1 = fcv[pl.ds(16, 16)]
    fc2 = fcv[pl.ds(32, 16)]
    fc3 = fcv[pl.ds(48, 16)]
    lane = lax.iota(jnp.int32, 16)

    def group(g, carry):
        wv = widx[pl.ds(g * _LANES, _LANES)]
        cv = cidx[pl.ds(g * _LANES, _LANES)]
        copies = []
        for r in range(_LANES):
            copies.append(pltpu.async_copy(emb_hbm.at[wv[r]], wrows.at[r], sem))
            copies.append(pltpu.async_copy(emb_hbm.at[cv[r]], crows.at[r], sem))
        for cp in copies:
            cp.wait()
        acc = jnp.zeros((_LANES,), jnp.float32)
        for r in range(_LANES):
            p = wrows[r, pl.ds(0, 16)] * crows[r, pl.ds(0, 16)] * fc0
            p = p + wrows[r, pl.ds(16, 16)] * crows[r, pl.ds(16, 16)] * fc1
            p = p + wrows[r, pl.ds(32, 16)] * crows[r, pl.ds(32, 16)] * fc2
            p = p + wrows[r, pl.ds(48, 16)] * crows[r, pl.ds(48, 16)] * fc3
            s = jnp.sum(p)
            acc = jnp.where(lane == r, s, acc)
        outv[pl.ds(g * _LANES, _LANES)] = acc
        return carry

    lax.fori_loop(0, _GROUPS, group, 0)

    pltpu.sync_copy(outv, out_hbm.at[pl.ds(base, _B_PER_W)])


def _bias_body(wids_hbm, cids_hbm, bias_hbm, z_hbm, out_hbm,
               widx, cidx, wbias, cbias, zv, sem):
    cid = lax.axis_index("c")
    sid = lax.axis_index("s")
    wid = sid * _NUM_CORES + cid
    base = wid * _B_PER_W

    pltpu.sync_copy(wids_hbm.at[pl.ds(base, _B_PER_W)], widx)
    pltpu.sync_copy(cids_hbm.at[pl.ds(base, _B_PER_W)], cidx)
    pltpu.sync_copy(z_hbm.at[pl.ds(base, _B_PER_W)], zv)

    copies = []
    for j in range(_N_CHUNKS):
        sl = pl.ds(j * _IDX_CHUNK, _IDX_CHUNK)
        copies.append(pltpu.async_copy(bias_hbm.at[widx.at[sl]], wbias.at[sl], sem))
        copies.append(pltpu.async_copy(bias_hbm.at[cidx.at[sl]], cbias.at[sl], sem))
    for cp in copies:
        cp.wait()

    def group(g, carry):
        sl = pl.ds(g * _LANES, _LANES)
        z = zv[sl] + wbias[sl] + cbias[sl]
        zv[sl] = 1.0 / (1.0 + jnp.exp(-z))
        return carry

    lax.fori_loop(0, _GROUPS, group, 0)
    pltpu.sync_copy(zv, out_hbm.at[pl.ds(base, _B_PER_W)])


@jax.jit
def _run(word_ids, context_ids, emb_table, bias_flat, fc_flat):
    mesh = plsc.VectorSubcoreMesh(core_axis_name="c", subcore_axis_name="s")
    zdot = functools.partial(
        pl.kernel,
        mesh=mesh,
        compiler_params=pltpu.CompilerParams(needs_layout_passes=False),
        out_type=jax.ShapeDtypeStruct((_BATCH,), jnp.float32),
        scratch_types=[
            pltpu.VMEM((_B_PER_W,), jnp.int32),
            pltpu.VMEM((_B_PER_W,), jnp.int32),
            pltpu.VMEM((_LANES, _EMB_DIM), jnp.float32),
            pltpu.VMEM((_LANES, _EMB_DIM), jnp.float32),
            pltpu.VMEM((_EMB_DIM,), jnp.float32),
            pltpu.VMEM((_B_PER_W,), jnp.float32),
            pltpu.SemaphoreType.DMA,
        ],
    )(_dot_body)(word_ids, context_ids, emb_table, fc_flat)

    out = functools.partial(
        pl.kernel,
        mesh=mesh,
        compiler_params=pltpu.CompilerParams(
            needs_layout_passes=False, use_tc_tiling_on_sc=False),
        out_type=jax.ShapeDtypeStruct((_BATCH,), jnp.float32),
        scratch_types=[
            pltpu.VMEM((_B_PER_W,), jnp.int32),
            pltpu.VMEM((_B_PER_W,), jnp.int32),
            pltpu.VMEM((_B_PER_W,), jnp.float32),
            pltpu.VMEM((_B_PER_W,), jnp.float32),
            pltpu.VMEM((_B_PER_W,), jnp.float32),
            pltpu.SemaphoreType.DMA,
        ],
    )(_bias_body)(word_ids, context_ids, bias_flat, zdot)
    return out


def kernel(word_ids, context_ids, emb_table, bias_table, fc_weight):
    word_ids = word_ids.astype(jnp.int32)
    context_ids = context_ids.astype(jnp.int32)
    bias_flat = bias_table.reshape(-1)
    fc_flat = fc_weight.reshape(-1)
    out = _run(word_ids, context_ids, emb_table, bias_flat, fc_flat)
    return out.reshape(_BATCH, 1)


# speedup vs baseline: 1.7420x; 1.1239x over previous
import functools

import jax
import jax.numpy as jnp
from jax import lax
from jax.experimental import pallas as pl
from jax.experimental.pallas import tpu as pltpu
from jax.experimental.pallas import tpu_sc as plsc

_NUM_CORES = 2
_NUM_SUBCORES = 16
_NUM_WORKERS = _NUM_CORES * _NUM_SUBCORES
_LANES = 16
_BATCH = 16384
_EMB_DIM = 64
_N_WORDS = 1000000
_B_PER_W = _BATCH // _NUM_WORKERS
_IDX_CHUNK = 128
_N_CHUNKS = _B_PER_W // _IDX_CHUNK
_GROUPS = _B_PER_W // _LANES

_NBLOCKS = (_N_WORDS + 127) // 128
_BLK_PER_W = (_NBLOCKS + _NUM_WORKERS - 1) // _NUM_WORKERS
_MATCH_CAP = 12288
_NSLOTS = 2 * _BATCH
_TRASH = _NSLOTS
_SCRATCH_ROWS = _NSLOTS + 8


def _gather_body(wids_hbm, cids_hbm, embt_hbm, scratch_hbm,
                 widx, cidx, matchbuf, binned, cnts, offs, curs,
                 bufs, flushbuf, slotbuf, sem, fsem):
    cid = lax.axis_index("c")
    sid = lax.axis_index("s")
    wid = sid * _NUM_CORES + cid
    lo = wid * _BLK_PER_W
    hi = jnp.minimum(lo + _BLK_PER_W, _NBLOCKS)
    nblk = hi - lo

    lane = lax.iota(jnp.int32, 16)
    lane0 = lane == 0
    zeros16 = jnp.zeros((16,), jnp.int32)

    pltpu.sync_copy(wids_hbm, widx)
    pltpu.sync_copy(cids_hbm, cidx)

    for t in range(16):
        cnts[pl.ds(t * 16, 16)] = zeros16
    for t in range(8):
        slotbuf[pl.ds(t * 16, 16)] = jnp.full((16,), _TRASH, jnp.int32)

    def scan_one(idx_ref, slot_base):
        def body(k, cnt):
            v = idx_ref[pl.ds(k * 16, 16)]
            blk = v >> 7
            m = (blk >= lo) & (blk < hi)
            blkl = jnp.where(m, blk - lo, 0)
            slot = slot_base + k * 16 + lane
            packed = (blkl << 22) | ((v & 127) << 15) | slot
            rank = plsc.cumsum(m.astype(jnp.int32))
            pos = jnp.minimum(cnt + rank - 1, _MATCH_CAP - 1)
            plsc.store_scatter(matchbuf, [pos], packed, mask=m)
            plsc.addupdate_scatter(cnts, [blkl], m.astype(jnp.int32), mask=m)
            return jnp.minimum(cnt + rank[15], _MATCH_CAP)
        return body

    cnt = lax.fori_loop(0, _BATCH // 16, scan_one(widx, 0), 0)
    cnt = lax.fori_loop(0, _BATCH // 16, scan_one(cidx, _BATCH), cnt)

    running = 0
    for q in range(16):
        c16 = cnts[pl.ds(q * 16, 16)]
        cs = plsc.cumsum(c16)
        offs[pl.ds(q * 16, 16)] = running + cs - c16
        curs[pl.ds(q * 16, 16)] = running + cs - c16
        running = running + cs[15]

    def place(g, carry):
        v = matchbuf[pl.ds(g * 16, 16)]
        for r in range(16):
            @pl.when(g * 16 + r < cnt)
            def _():
                item = v[r]
                blkl = item >> 22
                bsplat = jnp.full((16,), blkl, jnp.int32)
                cur = plsc.load_gather(curs, [bsplat])[0]
                plsc.store_scatter(
                    binned, [jnp.full((16,), cur, jnp.int32)],
                    jnp.full((16,), item, jnp.int32), mask=lane0)
                plsc.store_scatter(
                    curs, [bsplat],
                    jnp.full((16,), cur + 1, jnp.int32), mask=lane0)
        return carry

    lax.fori_loop(0, (cnt + 15) >> 4, place, 0)

    def fetch(j, p):
        col0 = pl.multiple_of((lo + j) * 128, 128)
        return pltpu.make_async_copy(
            embt_hbm.at[pl.ds(0, _EMB_DIM), pl.ds(col0, 128)],
            bufs.at[p], sem)

    fetch(0, 0).start()

    def flush():
        pltpu.sync_copy(flushbuf, scratch_hbm.at[slotbuf])
        for t in range(8):
            slotbuf[pl.ds(t * 16, 16)] = jnp.full((16,), _TRASH, jnp.int32)

    def block(j, fpos):
        p = j % 2

        @pl.when(j + 1 < nblk)
        def _():
            fetch(j + 1, (j + 1) % 2).start()

        fetch(j, p).wait()

        jsplat = jnp.full((16,), j, jnp.int32)
        n0 = plsc.load_gather(offs, [jsplat])[0]
        n1 = plsc.load_gather(curs, [jsplat])[0]

        def item(i, fp):
            it = plsc.load_gather(binned, [jnp.full((16,), i, jnp.int32)])[0]
            col = (it >> 15) & 127
            slot = it & 0x7FFF
            csplat = jnp.full((16,), col, jnp.int32)
            psplat = jnp.full((16,), p, jnp.int32)
            for q in range(4):
                vec = plsc.load_gather(
                    bufs, [psplat, lane + q * 16, csplat])
                flushbuf[fp, pl.ds(q * 16, 16)] = vec
            plsc.store_scatter(
                slotbuf, [jnp.full((16,), fp, jnp.int32)],
                jnp.full((16,), slot, jnp.int32), mask=lane0)
            fp = fp + 1

            @pl.when(fp == 128)
            def _():
                flush()

            return jnp.where(fp == 128, 0, fp)

        return lax.fori_loop(n0, n1, item, fpos)

    fpos = lax.fori_loop(0, nblk, block, 0)
    flush()


def _dot_body(scratch_hbm, fc_hbm, out_hbm, wrows, crows, fcv, outv, sem):
    cid = lax.axis_index("c")
    sid = lax.axis_index("s")
    wid = sid * _NUM_CORES + cid
    base = wid * _B_PER_W

    pltpu.sync_copy(fc_hbm, fcv)
    fc0 = fcv[pl.ds(0, 16)]
    fc1 = fcv[pl.ds(16, 16)]
    fc2 = fcv[pl.ds(32, 16)]
    fc3 = fcv[pl.ds(48, 16)]
    lane = lax.iota(jnp.int32, 16)

    half = _B_PER_W // 2
    for h in range(2):
        b0 = base + h * half
        cpw = pltpu.async_copy(scratch_hbm.at[pl.ds(b0, half)], wrows, sem)
        cpc = pltpu.async_copy(
            scratch_hbm.at[pl.ds(_BATCH + b0, half)], crows, sem)
        cpw.wait()
        cpc.wait()

        def group(g, carry):
            acc = jnp.zeros((_LANES,), jnp.float32)
            for r in range(_LANES):
                i = g * _LANES + r
                p = wrows[i, pl.ds(0, 16)] * crows[i, pl.ds(0, 16)] * fc0
                p = p + wrows[i, pl.ds(16, 16)] * crows[i, pl.ds(16, 16)] * fc1
                p = p + wrows[i, pl.ds(32, 16)] * crows[i, pl.ds(32, 16)] * fc2
                p = p + wrows[i, pl.ds(48, 16)] * crows[i, pl.ds(48, 16)] * fc3
                s = jnp.sum(p)
                acc = jnp.where(lane == r, s, acc)
            outv[pl.ds(h * half + g * _LANES, _LANES)] = acc
            return carry

        lax.fori_loop(0, half // _LANES, group, 0)

    pltpu.sync_copy(outv, out_hbm.at[pl.ds(base, _B_PER_W)])


def _bias_body(wids_hbm, cids_hbm, bias_hbm, z_hbm, out_hbm,
               widx, cidx, wbias, cbias, zv, sem):
    cid = lax.axis_index("c")
    sid = lax.axis_index("s")
    wid = sid * _NUM_CORES + cid
    base = wid * _B_PER_W

    pltpu.sync_copy(wids_hbm.at[pl.ds(base, _B_PER_W)], widx)
    pltpu.sync_copy(cids_hbm.at[pl.ds(base, _B_PER_W)], cidx)
    pltpu.sync_copy(z_hbm.at[pl.ds(base, _B_PER_W)], zv)

    copies = []
    for j in range(_N_CHUNKS):
        sl = pl.ds(j * _IDX_CHUNK, _IDX_CHUNK)
        copies.append(pltpu.async_copy(bias_hbm.at[widx.at[sl]], wbias.at[sl], sem))
        copies.append(pltpu.async_copy(bias_hbm.at[cidx.at[sl]], cbias.at[sl], sem))
    for cp in copies:
        cp.wait()

    def group(g, carry):
        sl = pl.ds(g * _LANES, _LANES)
        z = zv[sl] + wbias[sl] + cbias[sl]
        zv[sl] = 1.0 / (1.0 + jnp.exp(-z))
        return carry

    lax.fori_loop(0, _GROUPS, group, 0)
    pltpu.sync_copy(zv, out_hbm.at[pl.ds(base, _B_PER_W)])


@jax.jit
def _run(word_ids, context_ids, embt, bias_flat, fc_flat):
    mesh = plsc.VectorSubcoreMesh(core_axis_name="c", subcore_axis_name="s")
    tiled_params = pltpu.CompilerParams(needs_layout_passes=False)
    linear_params = pltpu.CompilerParams(
        needs_layout_passes=False, use_tc_tiling_on_sc=False)

    scratch = functools.partial(
        pl.kernel,
        mesh=mesh,
        compiler_params=tiled_params,
        out_type=jax.ShapeDtypeStruct((_SCRATCH_ROWS, 128), jnp.float32),
        scratch_types=[
            pltpu.VMEM((_BATCH,), jnp.int32),
            pltpu.VMEM((_BATCH,), jnp.int32),
            pltpu.VMEM((_MATCH_CAP,), jnp.int32),
            pltpu.VMEM((_MATCH_CAP,), jnp.int32),
            pltpu.VMEM((256,), jnp.int32),
            pltpu.VMEM((256,), jnp.int32),
            pltpu.VMEM((256,), jnp.int32),
            pltpu.VMEM((2, _EMB_DIM, 128), jnp.float32),
            pltpu.VMEM((128, 128), jnp.float32),
            pltpu.VMEM((128,), jnp.int32),
            pltpu.SemaphoreType.DMA,
            pltpu.SemaphoreType.DMA,
        ],
    )(_gather_body)(word_ids, context_ids, embt)

    zdot = functools.partial(
        pl.kernel,
        mesh=mesh,
        compiler_params=tiled_params,
        out_type=jax.ShapeDtypeStruct((_BATCH,), jnp.float32),
        scratch_types=[
            pltpu.VMEM((_B_PER_W // 2, 128), jnp.float32),
            pltpu.VMEM((_B_PER_W // 2, 128), jnp.float32),
            pltpu.VMEM((_EMB_DIM,), jnp.float32),
            pltpu.VMEM((_B_PER_W,), jnp.float32),
            pltpu.SemaphoreType.DMA,
        ],
    )(_dot_body)(scratch, fc_flat)

    out = functools.partial(
        pl.kernel,
        mesh=mesh,
        compiler_params=linear_params,
        out_type=jax.ShapeDtypeStruct((_BATCH,), jnp.float32),
        scratch_types=[
            pltpu.VMEM((_B_PER_W,), jnp.int32),
            pltpu.VMEM((_B_PER_W,), jnp.int32),
            pltpu.VMEM((_B_PER_W,), jnp.float32),
            pltpu.VMEM((_B_PER_W,), jnp.float32),
            pltpu.VMEM((_B_PER_W,), jnp.float32),
            pltpu.SemaphoreType.DMA,
        ],
    )(_bias_body)(word_ids, context_ids, bias_flat, zdot)
    return out


def kernel(word_ids, context_ids, emb_table, bias_table, fc_weight):
    word_ids = word_ids.astype(jnp.int32)
    context_ids = context_ids.astype(jnp.int32)
    bias_flat = bias_table.reshape(-1)
    fc_flat = fc_weight.reshape(-1)
    out = _run(word_ids, context_ids, emb_table.T, bias_flat, fc_flat)
    return out.reshape(_BATCH, 1)


# speedup vs baseline: 2.3394x; 1.3429x over previous
import functools

import jax
import jax.numpy as jnp
from jax import lax
from jax.experimental import pallas as pl
from jax.experimental.pallas import tpu as pltpu
from jax.experimental.pallas import tpu_sc as plsc

_NUM_CORES = 2
_NUM_SUBCORES = 16
_NUM_WORKERS = _NUM_CORES * _NUM_SUBCORES
_LANES = 16
_BATCH = 16384
_EMB_DIM = 64
_N_WORDS = 1000000
_B_PER_W = _BATCH // _NUM_WORKERS
_IDX_CHUNK = 128
_N_CHUNKS = _B_PER_W // _IDX_CHUNK
_GROUPS = _B_PER_W // _LANES

_NBLOCKS = (_N_WORDS + 127) // 128
_BLK_PER_W = (_NBLOCKS + _NUM_WORKERS - 1) // _NUM_WORKERS
_MATCH_CAP = 8192
_GRP = 4
_FLUSH = 64
_NSLOTS = 2 * _BATCH
_TRASH = _NSLOTS
_SCRATCH_ROWS = _NSLOTS + 8


def _gather_body(wids_hbm, cids_hbm, embt_hbm, scratch_hbm,
                 widx, cidx, matchbuf, binned, cnts, offs, curs,
                 bufs, flushbuf, slotbuf, sem, fsem):
    cid = lax.axis_index("c")
    sid = lax.axis_index("s")
    wid = sid * _NUM_CORES + cid
    lo = wid * _BLK_PER_W
    hi = jnp.minimum(lo + _BLK_PER_W, _NBLOCKS)
    nblk = hi - lo

    lane = lax.iota(jnp.int32, 16)
    lane0 = lane == 0
    zeros16 = jnp.zeros((16,), jnp.int32)

    pltpu.sync_copy(wids_hbm, widx)
    pltpu.sync_copy(cids_hbm, cidx)

    for t in range(16):
        cnts[pl.ds(t * 16, 16)] = zeros16
    for t in range(_FLUSH // 16):
        slotbuf[pl.ds(t * 16, 16)] = jnp.full((16,), _TRASH, jnp.int32)

    def scan_one(idx_ref, slot_base):
        def body(k, cnt):
            add = 0
            for u in range(4):
                kk = k * 4 + u
                v = idx_ref[pl.ds(kk * 16, 16)]
                blk = v >> 7
                m = (blk >= lo) & (blk < hi)
                blkl = jnp.where(m, blk - lo, 0)
                slot = slot_base + kk * 16 + lane
                packed = (blkl << 22) | ((v & 127) << 15) | slot
                rank = plsc.cumsum(m.astype(jnp.int32))
                pos = jnp.minimum(cnt + add + rank - 1, _MATCH_CAP - 1)
                plsc.store_scatter(matchbuf, [pos], packed, mask=m)
                plsc.addupdate_scatter(
                    cnts, [blkl], m.astype(jnp.int32), mask=m)
                add = add + rank[15]
            return jnp.minimum(cnt + add, _MATCH_CAP)
        return body

    cnt = lax.fori_loop(0, _BATCH // 64, scan_one(widx, 0), 0)
    cnt = lax.fori_loop(0, _BATCH // 64, scan_one(cidx, _BATCH), cnt)

    running = 0
    for q in range(16):
        c16 = cnts[pl.ds(q * 16, 16)]
        cs = plsc.cumsum(c16)
        offs[pl.ds(q * 16, 16)] = running + cs - c16
        curs[pl.ds(q * 16, 16)] = running + cs - c16
        running = running + cs[15]

    def place(g, carry):
        v = matchbuf[pl.ds(g * 16, 16)]
        for r in range(16):
            @pl.when(g * 16 + r < cnt)
            def _():
                item = v[r]
                blkl = item >> 22
                bsplat = jnp.full((16,), blkl, jnp.int32)
                cur = plsc.load_gather(curs, [bsplat])[0]
                plsc.store_scatter(
                    binned, [jnp.full((16,), cur, jnp.int32)],
                    jnp.full((16,), item, jnp.int32), mask=lane0)
                plsc.store_scatter(
                    curs, [bsplat],
                    jnp.full((16,), cur + 1, jnp.int32), mask=lane0)
        return carry

    lax.fori_loop(0, (cnt + 15) >> 4, place, 0)

    ngroups = (nblk + _GRP - 1) // _GRP

    def g0_of(g):
        return jnp.minimum(g * _GRP, nblk - _GRP)

    def fetch(g, p):
        col0 = pl.multiple_of((lo + g0_of(g)) * 128, 128)
        return pltpu.make_async_copy(
            embt_hbm.at[pl.ds(0, _EMB_DIM), pl.ds(col0, _GRP * 128)],
            bufs.at[p], sem)

    fetch(0, 0).start()

    def flush():
        pltpu.sync_copy(flushbuf, scratch_hbm.at[slotbuf])
        for t in range(_FLUSH // 16):
            slotbuf[pl.ds(t * 16, 16)] = jnp.full((16,), _TRASH, jnp.int32)

    def group_body(g, fpos):
        p = g % 2

        @pl.when(g + 1 < ngroups)
        def _():
            fetch(g + 1, (g + 1) % 2).start()

        fetch(g, p).wait()
        g0 = g0_of(g)

        for b in range(_GRP):
            j = g0 + b
            jsplat = jnp.full((16,), j, jnp.int32)
            n0 = plsc.load_gather(offs, [jsplat])[0]
            n1 = plsc.load_gather(curs, [jsplat])[0]

            def item(i, fp, b=b, p=p):
                it = plsc.load_gather(
                    binned, [jnp.full((16,), i, jnp.int32)])[0]
                col = b * 128 + ((it >> 15) & 127)
                slot = it & 0x7FFF
                csplat = jnp.full((16,), col, jnp.int32)
                psplat = jnp.full((16,), p, jnp.int32)
                for q in range(4):
                    vec = plsc.load_gather(
                        bufs, [psplat, lane + q * 16, csplat])
                    flushbuf[fp, pl.ds(q * 16, 16)] = vec
                plsc.store_scatter(
                    slotbuf, [jnp.full((16,), fp, jnp.int32)],
                    jnp.full((16,), slot, jnp.int32), mask=lane0)
                fp = fp + 1

                @pl.when(fp == _FLUSH)
                def _():
                    flush()

                return jnp.where(fp == _FLUSH, 0, fp)

            fpos = lax.fori_loop(n0, n1, item, fpos)
        return fpos

    fpos = lax.fori_loop(0, ngroups, group_body, 0)
    flush()


def _dot_body(scratch_hbm, fc_hbm, out_hbm, wrows, crows, fcv, outv, sem):
    cid = lax.axis_index("c")
    sid = lax.axis_index("s")
    wid = sid * _NUM_CORES + cid
    base = wid * _B_PER_W

    pltpu.sync_copy(fc_hbm, fcv)
    fc0 = fcv[pl.ds(0, 16)]
    fc1 = fcv[pl.ds(16, 16)]
    fc2 = fcv[pl.ds(32, 16)]
    fc3 = fcv[pl.ds(48, 16)]
    lane = lax.iota(jnp.int32, 16)

    half = _B_PER_W // 2
    for h in range(2):
        b0 = base + h * half
        cpw = pltpu.async_copy(scratch_hbm.at[pl.ds(b0, half)], wrows, sem)
        cpc = pltpu.async_copy(
            scratch_hbm.at[pl.ds(_BATCH + b0, half)], crows, sem)
        cpw.wait()
        cpc.wait()

        def group(g, carry):
            acc = jnp.zeros((_LANES,), jnp.float32)
            for r in range(_LANES):
                i = g * _LANES + r
                p = wrows[i, pl.ds(0, 16)] * crows[i, pl.ds(0, 16)] * fc0
                p = p + wrows[i, pl.ds(16, 16)] * crows[i, pl.ds(16, 16)] * fc1
                p = p + wrows[i, pl.ds(32, 16)] * crows[i, pl.ds(32, 16)] * fc2
                p = p + wrows[i, pl.ds(48, 16)] * crows[i, pl.ds(48, 16)] * fc3
                s = jnp.sum(p)
                acc = jnp.where(lane == r, s, acc)
            outv[pl.ds(h * half + g * _LANES, _LANES)] = acc
            return carry

        lax.fori_loop(0, half // _LANES, group, 0)

    pltpu.sync_copy(outv, out_hbm.at[pl.ds(base, _B_PER_W)])


def _bias_body(wids_hbm, cids_hbm, bias_hbm, z_hbm, out_hbm,
               widx, cidx, wbias, cbias, zv, sem):
    cid = lax.axis_index("c")
    sid = lax.axis_index("s")
    wid = sid * _NUM_CORES + cid
    base = wid * _B_PER_W

    pltpu.sync_copy(wids_hbm.at[pl.ds(base, _B_PER_W)], widx)
    pltpu.sync_copy(cids_hbm.at[pl.ds(base, _B_PER_W)], cidx)
    pltpu.sync_copy(z_hbm.at[pl.ds(base, _B_PER_W)], zv)

    copies = []
    for j in range(_N_CHUNKS):
        sl = pl.ds(j * _IDX_CHUNK, _IDX_CHUNK)
        copies.append(pltpu.async_copy(bias_hbm.at[widx.at[sl]], wbias.at[sl], sem))
        copies.append(pltpu.async_copy(bias_hbm.at[cidx.at[sl]], cbias.at[sl], sem))
    for cp in copies:
        cp.wait()

    def group(g, carry):
        sl = pl.ds(g * _LANES, _LANES)
        z = zv[sl] + wbias[sl] + cbias[sl]
        zv[sl] = 1.0 / (1.0 + jnp.exp(-z))
        return carry

    lax.fori_loop(0, _GROUPS, group, 0)
    pltpu.sync_copy(zv, out_hbm.at[pl.ds(base, _B_PER_W)])


@jax.jit
def _run(word_ids, context_ids, embt, bias_flat, fc_flat):
    mesh = plsc.VectorSubcoreMesh(core_axis_name="c", subcore_axis_name="s")
    tiled_params = pltpu.CompilerParams(needs_layout_passes=False)
    linear_params = pltpu.CompilerParams(
        needs_layout_passes=False, use_tc_tiling_on_sc=False)

    scratch = functools.partial(
        pl.kernel,
        mesh=mesh,
        compiler_params=tiled_params,
        out_type=jax.ShapeDtypeStruct((_SCRATCH_ROWS, 128), jnp.float32),
        scratch_types=[
            pltpu.VMEM((_BATCH,), jnp.int32),
            pltpu.VMEM((_BATCH,), jnp.int32),
            pltpu.VMEM((_MATCH_CAP,), jnp.int32),
            pltpu.VMEM((_MATCH_CAP,), jnp.int32),
            pltpu.VMEM((256,), jnp.int32),
            pltpu.VMEM((256,), jnp.int32),
            pltpu.VMEM((256,), jnp.int32),
            pltpu.VMEM((2, _EMB_DIM, _GRP * 128), jnp.float32),
            pltpu.VMEM((_FLUSH, 128), jnp.float32),
            pltpu.VMEM((_FLUSH,), jnp.int32),
            pltpu.SemaphoreType.DMA,
            pltpu.SemaphoreType.DMA,
        ],
    )(_gather_body)(word_ids, context_ids, embt)

    zdot = functools.partial(
        pl.kernel,
        mesh=mesh,
        compiler_params=tiled_params,
        out_type=jax.ShapeDtypeStruct((_BATCH,), jnp.float32),
        scratch_types=[
            pltpu.VMEM((_B_PER_W // 2, 128), jnp.float32),
            pltpu.VMEM((_B_PER_W // 2, 128), jnp.float32),
            pltpu.VMEM((_EMB_DIM,), jnp.float32),
            pltpu.VMEM((_B_PER_W,), jnp.float32),
            pltpu.SemaphoreType.DMA,
        ],
    )(_dot_body)(scratch, fc_flat)

    out = functools.partial(
        pl.kernel,
        mesh=mesh,
        compiler_params=linear_params,
        out_type=jax.ShapeDtypeStruct((_BATCH,), jnp.float32),
        scratch_types=[
            pltpu.VMEM((_B_PER_W,), jnp.int32),
            pltpu.VMEM((_B_PER_W,), jnp.int32),
            pltpu.VMEM((_B_PER_W,), jnp.float32),
            pltpu.VMEM((_B_PER_W,), jnp.float32),
            pltpu.VMEM((_B_PER_W,), jnp.float32),
            pltpu.SemaphoreType.DMA,
        ],
    )(_bias_body)(word_ids, context_ids, bias_flat, zdot)
    return out


def kernel(word_ids, context_ids, emb_table, bias_table, fc_weight):
    word_ids = word_ids.astype(jnp.int32)
    context_ids = context_ids.astype(jnp.int32)
    bias_flat = bias_table.reshape(-1)
    fc_flat = fc_weight.reshape(-1)
    out = _run(word_ids, context_ids, emb_table.T, bias_flat, fc_flat)
    return out.reshape(_BATCH, 1)


# speedup vs baseline: 2.6931x; 1.1512x over previous
import functools

import jax
import jax.numpy as jnp
from jax import lax
from jax.experimental import pallas as pl
from jax.experimental.pallas import tpu as pltpu
from jax.experimental.pallas import tpu_sc as plsc

_NUM_CORES = 2
_NUM_SUBCORES = 16
_NUM_WORKERS = _NUM_CORES * _NUM_SUBCORES
_LANES = 16
_BATCH = 16384
_EMB_DIM = 64
_N_WORDS = 1000000
_B_PER_W = _BATCH // _NUM_WORKERS
_IDX_CHUNK = 128
_N_CHUNKS = _B_PER_W // _IDX_CHUNK
_GROUPS = _B_PER_W // _LANES

_NBLOCKS = (_N_WORDS + 127) // 128
_BLK_PER_W = (_NBLOCKS + _NUM_WORKERS - 1) // _NUM_WORKERS
_MATCH_CAP = 8192
_GRP = 4
_FLUSH = 64
_NSLOTS = 2 * _BATCH
_TRASH = _NSLOTS
_SCRATCH_ROWS = _NSLOTS + 8


def _gather_body(wids_hbm, cids_hbm, embt_hbm, scratch_hbm,
                 widx, cidx, matchbuf, binned, cnts, offs, curs,
                 bufs, flushbuf, slotbuf, sem, fsem):
    cid = lax.axis_index("c")
    sid = lax.axis_index("s")
    wid = sid * _NUM_CORES + cid
    lo = wid * _BLK_PER_W
    hi = jnp.minimum(lo + _BLK_PER_W, _NBLOCKS)
    nblk = hi - lo

    lane = lax.iota(jnp.int32, 16)
    lane0 = lane == 0
    zeros16 = jnp.zeros((16,), jnp.int32)

    pltpu.sync_copy(wids_hbm, widx)
    pltpu.sync_copy(cids_hbm, cidx)

    ngroups = (nblk + _GRP - 1) // _GRP

    def g0_of(g):
        return jnp.minimum(g * _GRP, nblk - _GRP)

    def fetch(g, p):
        col0 = pl.multiple_of((lo + g0_of(g)) * 128, 128)
        return pltpu.make_async_copy(
            embt_hbm.at[pl.ds(0, _EMB_DIM), pl.ds(col0, _GRP * 128)],
            bufs.at[p], sem)

    fetch(0, 0).start()
    fetch(1, 1).start()

    for t in range(16):
        cnts[pl.ds(t * 16, 16)] = zeros16
    for t in range(_FLUSH // 16):
        slotbuf[pl.ds(t * 16, 16)] = jnp.full((16,), _TRASH, jnp.int32)

    def scan_one(idx_ref, slot_base):
        def body(k, cnt):
            add = 0
            for u in range(4):
                kk = k * 4 + u
                v = idx_ref[pl.ds(kk * 16, 16)]
                blk = v >> 7
                m = (blk >= lo) & (blk < hi)
                blkl = jnp.where(m, blk - lo, 0)
                slot = slot_base + kk * 16 + lane
                packed = (blkl << 22) | ((v & 127) << 15) | slot
                rank = plsc.cumsum(m.astype(jnp.int32))
                pos = jnp.minimum(cnt + add + rank - 1, _MATCH_CAP - 1)
                plsc.store_scatter(matchbuf, [pos], packed, mask=m)
                plsc.addupdate_scatter(
                    cnts, [blkl], m.astype(jnp.int32), mask=m)
                add = add + rank[15]
            return jnp.minimum(cnt + add, _MATCH_CAP)
        return body

    cnt = lax.fori_loop(0, _BATCH // 64, scan_one(widx, 0), 0)
    cnt = lax.fori_loop(0, _BATCH // 64, scan_one(cidx, _BATCH), cnt)

    running = 0
    for q in range(16):
        c16 = cnts[pl.ds(q * 16, 16)]
        cs = plsc.cumsum(c16)
        offs[pl.ds(q * 16, 16)] = running + cs - c16
        curs[pl.ds(q * 16, 16)] = running + cs - c16
        running = running + cs[15]

    def place(g, carry):
        v = matchbuf[pl.ds(g * 16, 16)]
        for r in range(16):
            @pl.when(g * 16 + r < cnt)
            def _():
                item = v[r]
                blkl = item >> 22
                bsplat = jnp.full((16,), blkl, jnp.int32)
                cur = plsc.load_gather(curs, [bsplat])[0]
                plsc.store_scatter(
                    binned, [jnp.full((16,), cur, jnp.int32)],
                    jnp.full((16,), item, jnp.int32), mask=lane0)
                plsc.store_scatter(
                    curs, [bsplat],
                    jnp.full((16,), cur + 1, jnp.int32), mask=lane0)
        return carry

    lax.fori_loop(0, (cnt + 15) >> 4, place, 0)

    def flush():
        pltpu.sync_copy(flushbuf, scratch_hbm.at[slotbuf])
        for t in range(_FLUSH // 16):
            slotbuf[pl.ds(t * 16, 16)] = jnp.full((16,), _TRASH, jnp.int32)

    def group_body(g, fpos):
        p = g % 2
        fetch(g, p).wait()
        g0 = g0_of(g)

        for b in range(_GRP):
            j = g0 + b
            jsplat = jnp.full((16,), j, jnp.int32)
            n0 = plsc.load_gather(offs, [jsplat])[0]
            n1 = plsc.load_gather(curs, [jsplat])[0]

            def item(i, fp, b=b, p=p):
                it = plsc.load_gather(
                    binned, [jnp.full((16,), i, jnp.int32)])[0]
                col = b * 128 + ((it >> 15) & 127)
                slot = it & 0x7FFF
                csplat = jnp.full((16,), col, jnp.int32)
                psplat = jnp.full((16,), p, jnp.int32)
                for q in range(4):
                    vec = plsc.load_gather(
                        bufs, [psplat, lane + q * 16, csplat])
                    flushbuf[fp, pl.ds(q * 16, 16)] = vec
                plsc.store_scatter(
                    slotbuf, [jnp.full((16,), fp, jnp.int32)],
                    jnp.full((16,), slot, jnp.int32), mask=lane0)
                fp = fp + 1

                @pl.when(fp == _FLUSH)
                def _():
                    flush()

                return jnp.where(fp == _FLUSH, 0, fp)

            fpos = lax.fori_loop(n0, n1, item, fpos)

        @pl.when(g + 2 < ngroups)
        def _():
            fetch(g + 2, p).start()

        return fpos

    fpos = lax.fori_loop(0, ngroups, group_body, 0)
    flush()


def _dot_body(wids_hbm, cids_hbm, scratch_hbm, bias_hbm, fc_hbm, out_hbm,
              widx, cidx, wbias, cbias, wrows, crows, fcv, outv, sem):
    cid = lax.axis_index("c")
    sid = lax.axis_index("s")
    wid = sid * _NUM_CORES + cid
    base = wid * _B_PER_W

    pltpu.sync_copy(wids_hbm.at[pl.ds(base, _B_PER_W)], widx)
    pltpu.sync_copy(cids_hbm.at[pl.ds(base, _B_PER_W)], cidx)
    bias_copies = []
    for j in range(_N_CHUNKS):
        sl = pl.ds(j * _IDX_CHUNK, _IDX_CHUNK)
        bias_copies.append(
            pltpu.async_copy(bias_hbm.at[widx.at[sl]], wbias.at[sl], sem))
        bias_copies.append(
            pltpu.async_copy(bias_hbm.at[cidx.at[sl]], cbias.at[sl], sem))

    pltpu.sync_copy(fc_hbm, fcv)
    fc0 = fcv[pl.ds(0, 16)]
    fc1 = fcv[pl.ds(16, 16)]
    fc2 = fcv[pl.ds(32, 16)]
    fc3 = fcv[pl.ds(48, 16)]
    lane = lax.iota(jnp.int32, 16)

    half = _B_PER_W // 2
    for h in range(2):
        b0 = base + h * half
        cpw = pltpu.async_copy(scratch_hbm.at[pl.ds(b0, half)], wrows, sem)
        cpc = pltpu.async_copy(
            scratch_hbm.at[pl.ds(_BATCH + b0, half)], crows, sem)
        cpw.wait()
        cpc.wait()

        def group(g, carry):
            acc = jnp.zeros((_LANES,), jnp.float32)
            for r in range(_LANES):
                i = g * _LANES + r
                p = wrows[i, pl.ds(0, 16)] * crows[i, pl.ds(0, 16)] * fc0
                p = p + wrows[i, pl.ds(16, 16)] * crows[i, pl.ds(16, 16)] * fc1
                p = p + wrows[i, pl.ds(32, 16)] * crows[i, pl.ds(32, 16)] * fc2
                p = p + wrows[i, pl.ds(48, 16)] * crows[i, pl.ds(48, 16)] * fc3
                s = jnp.sum(p)
                acc = jnp.where(lane == r, s, acc)
            outv[pl.ds(h * half + g * _LANES, _LANES)] = acc
            return carry

        lax.fori_loop(0, half // _LANES, group, 0)

    for cp in bias_copies:
        cp.wait()

    def final(g, carry):
        sl = pl.ds(g * _LANES, _LANES)
        z = outv[sl] + wbias[sl] + cbias[sl]
        outv[sl] = 1.0 / (1.0 + jnp.exp(-z))
        return carry

    lax.fori_loop(0, _GROUPS, final, 0)
    pltpu.sync_copy(outv, out_hbm.at[pl.ds(base, _B_PER_W)])


@jax.jit
def _run(word_ids, context_ids, embt, bias_flat, fc_flat):
    mesh = plsc.VectorSubcoreMesh(core_axis_name="c", subcore_axis_name="s")
    tiled_params = pltpu.CompilerParams(needs_layout_passes=False)

    scratch = functools.partial(
        pl.kernel,
        mesh=mesh,
        compiler_params=tiled_params,
        out_type=jax.ShapeDtypeStruct((_SCRATCH_ROWS, 128), jnp.float32),
        scratch_types=[
            pltpu.VMEM((_BATCH,), jnp.int32),
            pltpu.VMEM((_BATCH,), jnp.int32),
            pltpu.VMEM((_MATCH_CAP,), jnp.int32),
            pltpu.VMEM((_MATCH_CAP,), jnp.int32),
            pltpu.VMEM((256,), jnp.int32),
            pltpu.VMEM((256,), jnp.int32),
            pltpu.VMEM((256,), jnp.int32),
            pltpu.VMEM((2, _EMB_DIM, _GRP * 128), jnp.float32),
            pltpu.VMEM((_FLUSH, 128), jnp.float32),
            pltpu.VMEM((_FLUSH,), jnp.int32),
            pltpu.SemaphoreType.DMA,
            pltpu.SemaphoreType.DMA,
        ],
    )(_gather_body)(word_ids, context_ids, embt)

    out = functools.partial(
        pl.kernel,
        mesh=mesh,
        compiler_params=tiled_params,
        out_type=jax.ShapeDtypeStruct((_BATCH,), jnp.float32),
        scratch_types=[
            pltpu.VMEM((_B_PER_W,), jnp.int32),
            pltpu.VMEM((_B_PER_W,), jnp.int32),
            pltpu.VMEM((_B_PER_W,), jnp.float32),
            pltpu.VMEM((_B_PER_W,), jnp.float32),
            pltpu.VMEM((_B_PER_W // 2, 128), jnp.float32),
            pltpu.VMEM((_B_PER_W // 2, 128), jnp.float32),
            pltpu.VMEM((_EMB_DIM,), jnp.float32),
            pltpu.VMEM((_B_PER_W,), jnp.float32),
            pltpu.SemaphoreType.DMA,
        ],
    )(_dot_body)(word_ids, context_ids, scratch, bias_flat, fc_flat)
    return out


def kernel(word_ids, context_ids, emb_table, bias_table, fc_weight):
    word_ids = word_ids.astype(jnp.int32)
    context_ids = context_ids.astype(jnp.int32)
    bias_flat = bias_table.reshape(-1)
    fc_flat = fc_weight.reshape(-1)
    out = _run(word_ids, context_ids, emb_table.T, bias_flat, fc_flat)
    return out.reshape(_BATCH, 1)
